# baseline (speedup 1.0000x reference)
import jax
import jax.numpy as jnp
from jax.experimental import pallas as pl
from jax.experimental.pallas import tpu as pltpu

_BM = 256


def _body(adj_ref, emb_ref, out_ref):
    out_ref[...] = jnp.dot(adj_ref[...], emb_ref[...],
                           preferred_element_type=jnp.float32)


def kernel(adj, embeds):
    M, K = adj.shape
    _, N = embeds.shape
    return pl.pallas_call(
        _body,
        grid=(M // _BM,),
        in_specs=[
            pl.BlockSpec((_BM, K), lambda i: (i, 0)),
            pl.BlockSpec((K, N), lambda i: (0, 0)),
        ],
        out_specs=pl.BlockSpec((_BM, N), lambda i: (i, 0)),
        out_shape=jax.ShapeDtypeStruct((M, N), jnp.float32),
        compiler_params=pltpu.CompilerParams(
            dimension_semantics=("parallel",),
        ),
    )(adj, embeds)


# speedup vs baseline: 1.0557x; 1.0557x over previous
import jax
import jax.numpy as jnp
from jax.experimental import pallas as pl
from jax.experimental.pallas import tpu as pltpu

_BM = 1024


def _body(adj_ref, emb_ref, out_ref):
    out_ref[...] = jnp.dot(adj_ref[...], emb_ref[...],
                           preferred_element_type=jnp.float32)


def kernel(adj, embeds):
    M, K = adj.shape
    _, N = embeds.shape
    return pl.pallas_call(
        _body,
        grid=(M // _BM,),
        in_specs=[
            pl.BlockSpec((_BM, K), lambda i: (i, 0)),
            pl.BlockSpec((K, N), lambda i: (0, 0)),
        ],
        out_specs=pl.BlockSpec((_BM, N), lambda i: (i, 0)),
        out_shape=jax.ShapeDtypeStruct((M, N), jnp.float32),
        compiler_params=pltpu.CompilerParams(
            dimension_semantics=("parallel",),
        ),
    )(adj, embeds)
